# baseline (device time: 41758 ns/iter reference)
import jax
import jax.numpy as jnp
from jax import lax
from jax.experimental import pallas as pl
from jax.experimental.pallas import tpu as pltpu

N_DEV = 8
SQ = 512
D = 1024
DH = 128
HQ_LOC = 8
KV_LOC = 2
SCALE = 0.08838834764831843

_MASKS = (1, 3, 4)
HALF = 256
_PARTS = ((0, (0, 1, 2)), (HALF, (2, 0, 1)))
_RSBUF_OFF = (((0, 128), (128, 64), (192, 32)),
              ((224, 128), (352, 64), (416, 32)))


def kernel(x, Wq, Wo, Wk, Wv):
    def body(x_ref, wq_ref, wo_hbm, wk_hbm, wv_hbm, out_ref,
             p_ref, ag_ref, rsbuf_ref, wo_ref, wk_ref, wv_ref,
             load_sems, send_sems, recv_sems):
        my_i = lax.axis_index("i")
        q4 = my_i % 4
        bz = my_i // 4
        by = q4 // 2
        bx = (q4 % 2) ^ by
        bits = (bx, by, bz)
        partners = tuple((my_i ^ m) for m in _MASKS)

        kv_cols = pl.ds(my_i * (KV_LOC * DH), KV_LOC * DH)
        wk_cp = pltpu.make_async_copy(
            wk_hbm.at[:, kv_cols], wk_ref, load_sems.at[0])
        wv_cp = pltpu.make_async_copy(
            wv_hbm.at[:, kv_cols], wv_ref, load_sems.at[1])
        wo_cp = pltpu.make_async_copy(wo_hbm, wo_ref, load_sems.at[2])
        wk_cp.start()
        wv_cp.start()
        wo_cp.start()

        barrier_sem = pltpu.get_barrier_semaphore()
        for nbr in partners:
            pl.semaphore_signal(
                barrier_sem, inc=1,
                device_id=(nbr,), device_id_type=pl.DeviceIdType.MESH,
            )

        xv = x_ref[0, :, :].astype(jnp.bfloat16)
        qm = jnp.dot(xv, wq_ref[...].astype(jnp.bfloat16),
                     preferred_element_type=jnp.float32)
        wk_cp.wait()
        km = jnp.dot(xv, wk_ref[...].astype(jnp.bfloat16),
                     preferred_element_type=jnp.float32)
        wv_cp.wait()
        vm = jnp.dot(xv, wv_ref[...].astype(jnp.bfloat16),
                     preferred_element_type=jnp.float32)

        o_heads = []
        for h in range(HQ_LOC):
            qh = qm[:, h * DH:(h + 1) * DH].astype(jnp.bfloat16)
            kv = h // 4
            kh = km[:, kv * DH:(kv + 1) * DH].astype(jnp.bfloat16)
            vh = vm[:, kv * DH:(kv + 1) * DH].astype(jnp.bfloat16)
            s = jnp.dot(qh, kh.T, preferred_element_type=jnp.float32) * SCALE
            m = jnp.max(s, axis=-1, keepdims=True)
            p = jnp.exp(s - m).astype(jnp.bfloat16)
            l = jnp.sum(p, axis=-1, keepdims=True, dtype=jnp.float32)
            o_heads.append(
                jnp.dot(p, vh, preferred_element_type=jnp.float32) / l
            )
        o_loc = jnp.concatenate(o_heads, axis=1).astype(jnp.bfloat16)

        wo_cp.wait()
        p_ref[...] = jnp.dot(
            o_loc, wo_ref[...].astype(jnp.bfloat16),
            preferred_element_type=jnp.float32,
        ).astype(jnp.bfloat16)

        pl.semaphore_wait(barrier_sem, 3)

        off = [jnp.int32(0), jnp.int32(0)]
        pending = [None, None]

        def rs_start(ip, s):
            base, dims = _PARTS[ip]
            b = bits[dims[s]]
            seg = HALF >> (s + 1)
            send_off = base + off[ip] + (1 - b) * seg
            buf_off, _ = _RSBUF_OFF[ip][s]
            rdma = pltpu.make_async_remote_copy(
                src_ref=p_ref.at[pl.ds(send_off, seg), :],
                dst_ref=rsbuf_ref.at[pl.ds(buf_off, seg), :],
                send_sem=send_sems.at[ip * 6 + s],
                recv_sem=recv_sems.at[ip * 6 + s],
                device_id=(partners[dims[s]],),
                device_id_type=pl.DeviceIdType.MESH,
            )
            rdma.start()
            pending[ip] = ("rs", rdma, s)

        def ag_start(ip, s):
            base, dims = _PARTS[ip]
            seg = 32 << s
            src = ag_ref.at[pl.ds(base + off[ip], seg), :]
            rdma = pltpu.make_async_remote_copy(
                src_ref=src,
                dst_ref=src,
                send_sem=send_sems.at[ip * 6 + 3 + s],
                recv_sem=recv_sems.at[ip * 6 + 3 + s],
                device_id=(partners[dims[2 - s]],),
                device_id_type=pl.DeviceIdType.MESH,
            )
            rdma.start()
            pending[ip] = ("ag", rdma, s)

        def finish(ip):
            if pending[ip] is None:
                return
            kind, rdma, s = pending[ip]
            pending[ip] = None
            base, dims = _PARTS[ip]
            rdma.wait()
            if kind == "rs":
                b = bits[dims[s]]
                seg = HALF >> (s + 1)
                buf_off, _ = _RSBUF_OFF[ip][s]
                keep = base + off[ip] + b * seg
                p_ref[pl.ds(keep, seg), :] = (
                    p_ref[pl.ds(keep, seg), :]
                    + rsbuf_ref[pl.ds(buf_off, seg), :]
                )
                off[ip] = off[ip] + b * seg
                if s == 2:
                    ag_ref[pl.ds(base + off[ip], 32), :] = (
                        p_ref[pl.ds(base + off[ip], 32), :]
                    )
            else:
                b = bits[dims[2 - s]]
                seg = 32 << s
                off[ip] = off[ip] - b * seg

        for step in range(6):
            for ip in range(2):
                finish(ip)
                if step < 3:
                    rs_start(ip, step)
                else:
                    ag_start(ip, step - 3)
        for ip in range(2):
            finish(ip)

        out_ref[0, :, :] = ag_ref[...].astype(jnp.float32)

    out = pl.pallas_call(
        body,
        out_shape=jax.ShapeDtypeStruct((1, SQ, D), jnp.float32),
        in_specs=[
            pl.BlockSpec(memory_space=pltpu.VMEM),
            pl.BlockSpec(memory_space=pltpu.VMEM),
            pl.BlockSpec(memory_space=pl.ANY),
            pl.BlockSpec(memory_space=pl.ANY),
            pl.BlockSpec(memory_space=pl.ANY),
        ],
        out_specs=pl.BlockSpec(memory_space=pltpu.VMEM),
        scratch_shapes=[
            pltpu.VMEM((SQ, D), jnp.bfloat16),
            pltpu.VMEM((SQ, D), jnp.bfloat16),
            pltpu.VMEM((448, D), jnp.bfloat16),
            pltpu.VMEM((D, D), jnp.float32),
            pltpu.VMEM((D, KV_LOC * DH), jnp.float32),
            pltpu.VMEM((D, KV_LOC * DH), jnp.float32),
            pltpu.SemaphoreType.DMA((3,)),
            pltpu.SemaphoreType.DMA((12,)),
            pltpu.SemaphoreType.DMA((12,)),
        ],
        compiler_params=pltpu.CompilerParams(collective_id=0),
    )(x, Wq, Wo, Wk, Wv)
    return out


# device time: 37351 ns/iter; 1.1180x vs baseline; 1.1180x over previous
import jax
import jax.numpy as jnp
from jax import lax
from jax.experimental import pallas as pl
from jax.experimental.pallas import tpu as pltpu

N_DEV = 8
SQ = 512
D = 1024
DH = 128
HQ_LOC = 8
KV_LOC = 2
SCALE = 0.08838834764831843

_MASKS = (1, 3, 4)
HALF = 256
_PARTS = ((0, (0, 1, 2)), (HALF, (2, 0, 1)))
_RSBUF_OFF = (((0, 128), (128, 64), (192, 64)),
              ((256, 128), (384, 64), (448, 64)))


def kernel(x, Wq, Wo, Wk, Wv):
    my = lax.axis_index("i")
    wk_loc = lax.dynamic_slice(Wk, (0, my * (KV_LOC * DH)), (D, KV_LOC * DH))
    wv_loc = lax.dynamic_slice(Wv, (0, my * (KV_LOC * DH)), (D, KV_LOC * DH))

    def body(x_ref, wq_ref, wo_ref, wk_ref, wv_ref, out_ref,
             p_ref, rsbuf_ref, send_sems, recv_sems):
        my_i = lax.axis_index("i")
        q4 = my_i % 4
        bz = my_i // 4
        by = q4 // 2
        bx = (q4 % 2) ^ by
        bits = (bx, by, bz)
        partners = tuple((my_i ^ m) for m in _MASKS)

        barrier_sem = pltpu.get_barrier_semaphore()
        for nbr in partners:
            pl.semaphore_signal(
                barrier_sem, inc=1,
                device_id=(nbr,), device_id_type=pl.DeviceIdType.MESH,
            )

        xv = x_ref[0, :, :].astype(jnp.bfloat16)
        qm = jnp.dot(xv, wq_ref[...].astype(jnp.bfloat16),
                     preferred_element_type=jnp.float32)
        km = jnp.dot(xv, wk_ref[...].astype(jnp.bfloat16),
                     preferred_element_type=jnp.float32)
        vm = jnp.dot(xv, wv_ref[...].astype(jnp.bfloat16),
                     preferred_element_type=jnp.float32)

        o_heads = []
        for h in range(HQ_LOC):
            qh = qm[:, h * DH:(h + 1) * DH].astype(jnp.bfloat16)
            kv = h // 4
            kh = km[:, kv * DH:(kv + 1) * DH].astype(jnp.bfloat16)
            vh = vm[:, kv * DH:(kv + 1) * DH].astype(jnp.bfloat16)
            s = jnp.dot(qh, kh.T, preferred_element_type=jnp.float32) * SCALE
            m = jnp.max(s, axis=-1, keepdims=True)
            p = jnp.exp(s - m).astype(jnp.bfloat16)
            l = jnp.sum(p, axis=-1, keepdims=True, dtype=jnp.float32)
            o_heads.append(
                jnp.dot(p, vh, preferred_element_type=jnp.float32) / l
            )
        o_loc = jnp.concatenate(o_heads, axis=1).astype(jnp.bfloat16)

        p_ref[...] = jnp.dot(
            o_loc, wo_ref[...].astype(jnp.bfloat16),
            preferred_element_type=jnp.float32,
        ).astype(jnp.bfloat16)

        pl.semaphore_wait(barrier_sem, 3)

        off = [jnp.int32(0), jnp.int32(0)]
        size = [HALF, HALF]
        pending = [None, None]

        def red_start(ip, s):
            base, dims = _PARTS[ip]
            b = bits[dims[s]]
            if s < 2:
                seg = size[ip] // 2
                send_off = base + off[ip] + (1 - b) * seg
            else:
                seg = size[ip]
                send_off = base + off[ip]
            buf_off, _ = _RSBUF_OFF[ip][s]
            rdma = pltpu.make_async_remote_copy(
                src_ref=p_ref.at[pl.ds(send_off, seg), :],
                dst_ref=rsbuf_ref.at[pl.ds(buf_off, seg), :],
                send_sem=send_sems.at[ip * 5 + s],
                recv_sem=recv_sems.at[ip * 5 + s],
                device_id=(partners[dims[s]],),
                device_id_type=pl.DeviceIdType.MESH,
            )
            rdma.start()
            pending[ip] = ("red", rdma, s)

        def ag_start(ip, s):
            base, dims = _PARTS[ip]
            seg = size[ip]
            src = p_ref.at[pl.ds(base + off[ip], seg), :]
            rdma = pltpu.make_async_remote_copy(
                src_ref=src,
                dst_ref=src,
                send_sem=send_sems.at[ip * 5 + s],
                recv_sem=recv_sems.at[ip * 5 + s],
                device_id=(partners[dims[4 - s]],),
                device_id_type=pl.DeviceIdType.MESH,
            )
            rdma.start()
            pending[ip] = ("ag", rdma, s)

        def finish(ip):
            if pending[ip] is None:
                return
            kind, rdma, s = pending[ip]
            pending[ip] = None
            base, dims = _PARTS[ip]
            rdma.wait()
            if kind == "red":
                b = bits[dims[s]]
                buf_off, _ = _RSBUF_OFF[ip][s]
                if s < 2:
                    seg = size[ip] // 2
                    keep = off[ip] + b * seg
                    size[ip] = seg
                    off[ip] = keep
                else:
                    seg = size[ip]
                    keep = off[ip]
                p_ref[pl.ds(base + keep, seg), :] = (
                    p_ref[pl.ds(base + keep, seg), :]
                    + rsbuf_ref[pl.ds(buf_off, seg), :]
                )
            else:
                b = bits[dims[4 - s]]
                size2 = size[ip] * 2
                off[ip] = off[ip] - b * size[ip]
                size[ip] = size2

        for step in range(5):
            for ip in range(2):
                finish(ip)
                if step < 3:
                    red_start(ip, step)
                else:
                    ag_start(ip, step)
        for ip in range(2):
            finish(ip)

        out_ref[0, :, :] = p_ref[...]

    out = pl.pallas_call(
        body,
        out_shape=jax.ShapeDtypeStruct((1, SQ, D), jnp.bfloat16),
        in_specs=[pl.BlockSpec(memory_space=pltpu.VMEM)] * 5,
        out_specs=pl.BlockSpec(memory_space=pltpu.VMEM),
        scratch_shapes=[
            pltpu.VMEM((SQ, D), jnp.bfloat16),
            pltpu.VMEM((SQ, D), jnp.bfloat16),
            pltpu.SemaphoreType.DMA((10,)),
            pltpu.SemaphoreType.DMA((10,)),
        ],
        compiler_params=pltpu.CompilerParams(collective_id=0),
    )(x, Wq, Wo, wk_loc, wv_loc)
    return out
